# baseline (speedup 1.0000x reference)
import functools

import jax
import jax.numpy as jnp
from jax import lax
from jax.experimental import pallas as pl
from jax.experimental.pallas import tpu as pltpu
from jax.experimental.pallas import tpu_sc as plsc

NC = 2
NS = 16
NW = NC * NS
L = 16
CHUNK = 80
NB = 4


def _sc_mesh():
    return plsc.VectorSubcoreMesh(core_axis_name="c", subcore_axis_name="s")


def _deg_partials(col3, c3, ndp):
    n_chunks = col3.shape[1]
    zpt = ndp // NS

    @functools.partial(
        pl.kernel,
        out_type=jax.ShapeDtypeStruct((NC, 1, ndp), jnp.float32),
        mesh=_sc_mesh(),
        scratch_types=[
            pltpu.VMEM((n_chunks, CHUNK), jnp.int32),
            pltpu.VMEM((n_chunks, CHUNK), jnp.float32),
            pltpu.VMEM((zpt,), jnp.float32),
            pltpu.VMEM_SHARED((ndp,), jnp.float32),
        ],
    )
    def k(col_hbm, c_hbm, out_hbm, col2d, c2d, zbuf, deg_acc):
        cid = lax.axis_index("c")
        sid = lax.axis_index("s")
        wid = cid * NS + sid
        zero16 = jnp.zeros((L,), jnp.float32)

        def zz(i, carry):
            zbuf[pl.ds(i * L, L)] = zero16
            return carry

        lax.fori_loop(0, zpt // L, zz, 0)
        pltpu.sync_copy(zbuf, deg_acc.at[pl.ds(sid * zpt, zpt)])
        pltpu.sync_copy(col_hbm.at[wid], col2d)
        pltpu.sync_copy(c_hbm.at[wid], c2d)
        plsc.subcore_barrier()

        def body(i, carry):
            pltpu.sync_copy(c2d.at[i], deg_acc.at[col2d.at[i]], add=True)
            return carry

        lax.fori_loop(0, n_chunks, body, 0)
        plsc.subcore_barrier()
        pltpu.sync_copy(deg_acc.at[pl.ds(sid * zpt, zpt)],
                        out_hbm.at[cid, 0, pl.ds(sid * zpt, zpt)])

    return k(col3, c3)


def _dis_from_deg(degp):
    def body(deg_ref, out_ref):
        d = jnp.sum(deg_ref[...], axis=0)
        out_ref[...] = jnp.where(
            d > 0, lax.rsqrt(jnp.maximum(d, 1e-30)), 0.0)

    return pl.pallas_call(
        body,
        out_shape=jax.ShapeDtypeStruct(degp.shape[1:], jnp.float32),
    )(degp)


def _prop_partials(x, colp, rowp, cp, disf, n_chunks, n, d):
    rpt = n // NS
    zrows = CHUNK
    ndp = disf.shape[0]

    @functools.partial(
        pl.kernel,
        out_type=jax.ShapeDtypeStruct((NC, n, d), jnp.float32),
        mesh=_sc_mesh(),
        scratch_types=(
            [pltpu.VMEM((NB, CHUNK), jnp.int32),
             pltpu.VMEM((NB, CHUNK), jnp.int32),
             pltpu.VMEM((NB, CHUNK), jnp.float32),
             pltpu.VMEM((NB, CHUNK), jnp.float32),
             pltpu.VMEM((NB, CHUNK), jnp.float32)]
            + [pltpu.VMEM((CHUNK, 128), jnp.float32) for _ in range(NB)]
            + [pltpu.VMEM_SHARED((n, d), jnp.float32)]
            + [pltpu.SemaphoreType.DMA for _ in range(2 * NB)]
        ),
    )
    def k(x_hbm, col_hbm, row_hbm, c_hbm, dis_hbm, out_hbm, *refs):
        refs = list(refs)
        colv, rowv, cvb, dcv, drv = refs[:5]
        rbufs = refs[5:5 + NB]
        acc = refs[5 + NB]
        esems = refs[6 + NB:6 + 2 * NB]
        sems = refs[6 + 2 * NB:6 + 3 * NB]
        cid = lax.axis_index("c")
        sid = lax.axis_index("s")
        wid = cid * NS + sid
        zero16 = jnp.zeros((L,), jnp.float32)
        ept = n_chunks * CHUNK

        def zr(i, carry):
            for j in range(d // L):
                rbufs[0][i, pl.ds(L * j, L)] = zero16
            return carry

        lax.fori_loop(0, zrows, zr, 0)

        def zacc(i, carry):
            pltpu.sync_copy(rbufs[0],
                            acc.at[pl.ds(sid * rpt + i * zrows, zrows)])
            return carry

        lax.fori_loop(0, rpt // zrows, zacc, 0)
        rem = rpt % zrows
        if rem:
            pltpu.sync_copy(
                rbufs[0].at[pl.ds(0, rem)],
                acc.at[pl.ds(sid * rpt + (rpt // zrows) * zrows, rem)])
        plsc.subcore_barrier()

        def start_edges(i, buf):
            base = wid * ept + i * CHUNK
            pltpu.async_copy(col_hbm.at[pl.ds(base, CHUNK)],
                             colv.at[buf], esems[buf])
            pltpu.async_copy(row_hbm.at[pl.ds(base, CHUNK)],
                             rowv.at[buf], esems[buf])
            pltpu.async_copy(c_hbm.at[pl.ds(base, CHUNK)],
                             cvb.at[buf], esems[buf])

        def wait_edges(i, buf):
            base = wid * ept + i * CHUNK
            pltpu.make_async_copy(col_hbm.at[pl.ds(base, CHUNK)],
                                  colv.at[buf], esems[buf]).wait()
            pltpu.make_async_copy(row_hbm.at[pl.ds(base, CHUNK)],
                                  rowv.at[buf], esems[buf]).wait()
            pltpu.make_async_copy(c_hbm.at[pl.ds(base, CHUNK)],
                                  cvb.at[buf], esems[buf]).wait()

        def start_gathers(buf):
            pltpu.async_copy(x_hbm.at[colv.at[buf]], rbufs[buf], sems[buf])
            pltpu.async_copy(dis_hbm.at[colv.at[buf]], dcv.at[buf], sems[buf])
            pltpu.async_copy(dis_hbm.at[rowv.at[buf]], drv.at[buf], sems[buf])

        def wait_gathers(buf):
            pltpu.make_async_copy(x_hbm.at[colv.at[buf]],
                                  rbufs[buf], sems[buf]).wait()
            pltpu.make_async_copy(dis_hbm.at[colv.at[buf]],
                                  dcv.at[buf], sems[buf]).wait()
            pltpu.make_async_copy(dis_hbm.at[rowv.at[buf]],
                                  drv.at[buf], sems[buf]).wait()

        for q in range(NB - 1):
            start_edges(q, q)
            wait_edges(q, q)
            start_gathers(q)
        start_edges(NB - 1, NB - 1)

        def outer(io, carry):
            for b in range(NB):
                i = io * NB + b
                b2 = (b + NB - 1) % NB

                @pl.when(i + NB - 1 < n_chunks)
                def _():
                    wait_edges(i + NB - 1, b2)
                    start_gathers(b2)

                wait_gathers(b)
                rows = rbufs[b]

                def scale(g, c2):
                    svec = (cvb[b, pl.ds(g * L, L)]
                            * dcv[b, pl.ds(g * L, L)]
                            * drv[b, pl.ds(g * L, L)])
                    for kq in range(L):
                        s = svec[kq]
                        e2 = g * L + kq
                        for j in range(d // L):
                            rows[e2, pl.ds(L * j, L)] = (
                                rows[e2, pl.ds(L * j, L)] * s)
                    return c2

                lax.fori_loop(0, CHUNK // L, scale, 0)
                pltpu.sync_copy(rows, acc.at[rowv.at[b]], add=True)

                @pl.when(i + NB < n_chunks)
                def _():
                    start_edges(i + NB, b)
            return carry

        lax.fori_loop(0, n_chunks // NB, outer, 0)
        plsc.subcore_barrier()
        pltpu.sync_copy(acc.at[pl.ds(sid * rpt, rpt)],
                        out_hbm.at[cid, pl.ds(sid * rpt, rpt)])

    return k(x, colp, rowp, cp, disf)


def _linear(p, wt, b2, n, d):
    r = n // 8

    def body(p_ref, w_ref, b_ref, out_ref):
        y = p_ref[0] + p_ref[1]
        out_ref[...] = (
            jnp.dot(y, w_ref[...], preferred_element_type=jnp.float32)
            + b_ref[...])

    return pl.pallas_call(
        body,
        grid=(8,),
        in_specs=[
            pl.BlockSpec((NC, r, d), lambda i: (0, i, 0)),
            pl.BlockSpec((d, d), lambda i: (0, 0)),
            pl.BlockSpec((1, d), lambda i: (0, 0)),
        ],
        out_specs=pl.BlockSpec((r, d), lambda i: (i, 0)),
        out_shape=jax.ShapeDtypeStruct((n, d), jnp.float32),
    )(p, wt, b2)


def kernel(x, edge_index, C, W, b):
    n, d = x.shape
    e = C.shape[0]
    row = edge_index[0]
    col = edge_index[1]

    block = NW * CHUNK
    n_chunks = -(-e // block)
    n_chunks = -(-n_chunks // NB) * NB
    pad = n_chunks * block - e
    if pad:
        zi = jnp.zeros((pad,), jnp.int32)
        row = jnp.concatenate([row, zi])
        col = jnp.concatenate([col, zi])
        cp = jnp.concatenate([C, jnp.zeros((pad,), jnp.float32)])
    else:
        cp = C
    col3 = col.reshape(NW, n_chunks, CHUNK)
    row3 = row.reshape(NW, n_chunks, CHUNK)
    c3 = cp.reshape(NW, n_chunks, CHUNK)

    drows = -(-n // 128)
    drows = -(-drows // NS) * NS
    np_pad = drows * 128
    degp = _deg_partials(col3, c3, np_pad)
    dis = _dis_from_deg(degp.reshape(NC, drows, 128))
    disf = dis.reshape(np_pad)
    np_acc = -(-n // 128) * 128
    p = _prop_partials(x, col, row, cp, disf, n_chunks, np_acc, d)
    out = _linear(p, W.T, b.reshape(1, d), np_acc, d)
    return out[:n]


# speedup vs baseline: 1.1174x; 1.1174x over previous
import functools

import jax
import jax.numpy as jnp
from jax import lax
from jax.experimental import pallas as pl
from jax.experimental.pallas import tpu as pltpu
from jax.experimental.pallas import tpu_sc as plsc

NC = 2
NS = 16
NW = NC * NS
L = 16
CHUNK = 64
NB = 3


def _sc_mesh():
    return plsc.VectorSubcoreMesh(core_axis_name="c", subcore_axis_name="s")


def _deg_partials(col3, c3, ndp):
    n_chunks = col3.shape[1]
    zpt = ndp // NS

    @functools.partial(
        pl.kernel,
        out_type=jax.ShapeDtypeStruct((NC, 1, ndp), jnp.float32),
        mesh=_sc_mesh(),
        scratch_types=[
            pltpu.VMEM((n_chunks, CHUNK), jnp.int32),
            pltpu.VMEM((n_chunks, CHUNK), jnp.float32),
            pltpu.VMEM((zpt,), jnp.float32),
            pltpu.VMEM_SHARED((ndp,), jnp.float32),
        ],
    )
    def k(col_hbm, c_hbm, out_hbm, col2d, c2d, zbuf, deg_acc):
        cid = lax.axis_index("c")
        sid = lax.axis_index("s")
        wid = cid * NS + sid
        zero16 = jnp.zeros((L,), jnp.float32)

        def zz(i, carry):
            zbuf[pl.ds(i * L, L)] = zero16
            return carry

        lax.fori_loop(0, zpt // L, zz, 0)
        pltpu.sync_copy(zbuf, deg_acc.at[pl.ds(sid * zpt, zpt)])
        pltpu.sync_copy(col_hbm.at[wid], col2d)
        pltpu.sync_copy(c_hbm.at[wid], c2d)
        plsc.subcore_barrier()

        def body(i, carry):
            pltpu.sync_copy(c2d.at[i], deg_acc.at[col2d.at[i]], add=True)
            return carry

        lax.fori_loop(0, n_chunks, body, 0)
        plsc.subcore_barrier()
        pltpu.sync_copy(deg_acc.at[pl.ds(sid * zpt, zpt)],
                        out_hbm.at[cid, 0, pl.ds(sid * zpt, zpt)])

    return k(col3, c3)


def _dis_from_deg(degp):
    def body(deg_ref, out_ref):
        d = jnp.sum(deg_ref[...], axis=0)
        out_ref[...] = jnp.where(
            d > 0, lax.rsqrt(jnp.maximum(d, 1e-30)), 0.0)

    return pl.pallas_call(
        body,
        out_shape=jax.ShapeDtypeStruct(degp.shape[1:], jnp.float32),
    )(degp)


def _prop_partials(x, colp, rowp, cp, disf, n_chunks, n, d):
    rpt = n // NS
    zrows = CHUNK
    ndp = disf.shape[0]

    @functools.partial(
        pl.kernel,
        out_type=jax.ShapeDtypeStruct((NC, n, d), jnp.float32),
        mesh=_sc_mesh(),
        scratch_types=(
            [pltpu.VMEM((NB, CHUNK), jnp.int32),
             pltpu.VMEM((NB, CHUNK), jnp.int32),
             pltpu.VMEM((NB, CHUNK), jnp.float32),
             pltpu.VMEM((NB, CHUNK), jnp.float32),
             pltpu.VMEM((NB, CHUNK), jnp.float32)]
            + [pltpu.VMEM((CHUNK, 128), jnp.float32) for _ in range(NB)]
            + [pltpu.VMEM_SHARED((n, d), jnp.float32)]
            + [pltpu.SemaphoreType.DMA for _ in range(2 * NB)]
        ),
    )
    def k(x_hbm, col_hbm, row_hbm, c_hbm, dis_hbm, out_hbm, *refs):
        refs = list(refs)
        colv, rowv, cvb, dcv, drv = refs[:5]
        rbufs = refs[5:5 + NB]
        acc = refs[5 + NB]
        esems = refs[6 + NB:6 + 2 * NB]
        sems = refs[6 + 2 * NB:6 + 3 * NB]
        cid = lax.axis_index("c")
        sid = lax.axis_index("s")
        wid = cid * NS + sid
        zero16 = jnp.zeros((L,), jnp.float32)
        ept = n_chunks * CHUNK

        def zr(i, carry):
            for j in range(d // L):
                rbufs[0][i, pl.ds(L * j, L)] = zero16
            return carry

        lax.fori_loop(0, zrows, zr, 0)

        def zacc(i, carry):
            pltpu.sync_copy(rbufs[0],
                            acc.at[pl.ds(sid * rpt + i * zrows, zrows)])
            return carry

        lax.fori_loop(0, rpt // zrows, zacc, 0)
        rem = rpt % zrows
        if rem:
            pltpu.sync_copy(
                rbufs[0].at[pl.ds(0, rem)],
                acc.at[pl.ds(sid * rpt + (rpt // zrows) * zrows, rem)])
        plsc.subcore_barrier()

        def start_edges(i, buf):
            base = wid * ept + i * CHUNK
            pltpu.async_copy(col_hbm.at[pl.ds(base, CHUNK)],
                             colv.at[buf], esems[buf])
            pltpu.async_copy(row_hbm.at[pl.ds(base, CHUNK)],
                             rowv.at[buf], esems[buf])
            pltpu.async_copy(c_hbm.at[pl.ds(base, CHUNK)],
                             cvb.at[buf], esems[buf])

        def wait_edges(i, buf):
            base = wid * ept + i * CHUNK
            pltpu.make_async_copy(col_hbm.at[pl.ds(base, CHUNK)],
                                  colv.at[buf], esems[buf]).wait()
            pltpu.make_async_copy(row_hbm.at[pl.ds(base, CHUNK)],
                                  rowv.at[buf], esems[buf]).wait()
            pltpu.make_async_copy(c_hbm.at[pl.ds(base, CHUNK)],
                                  cvb.at[buf], esems[buf]).wait()

        def start_gathers(buf):
            pltpu.async_copy(x_hbm.at[colv.at[buf]], rbufs[buf], sems[buf])
            pltpu.async_copy(dis_hbm.at[colv.at[buf]], dcv.at[buf], sems[buf])
            pltpu.async_copy(dis_hbm.at[rowv.at[buf]], drv.at[buf], sems[buf])

        def wait_gathers(buf):
            pltpu.make_async_copy(x_hbm.at[colv.at[buf]],
                                  rbufs[buf], sems[buf]).wait()
            pltpu.make_async_copy(dis_hbm.at[colv.at[buf]],
                                  dcv.at[buf], sems[buf]).wait()
            pltpu.make_async_copy(dis_hbm.at[rowv.at[buf]],
                                  drv.at[buf], sems[buf]).wait()

        start_edges(0, 0)
        wait_edges(0, 0)
        start_gathers(0)
        start_edges(1, 1)
        wait_edges(1, 1)
        start_gathers(1)
        start_edges(2, 2)

        def outer(io, carry):
            for b in range(NB):
                i = io * NB + b
                b2 = (b + 2) % NB

                @pl.when(i + 2 < n_chunks)
                def _():
                    wait_edges(i + 2, b2)
                    start_gathers(b2)

                wait_gathers(b)
                rows = rbufs[b]

                def scale(g, c2):
                    svec = (cvb[b, pl.ds(g * L, L)]
                            * dcv[b, pl.ds(g * L, L)]
                            * drv[b, pl.ds(g * L, L)])
                    for kq in range(L):
                        s = svec[kq]
                        e2 = g * L + kq
                        for j in range(d // L):
                            rows[e2, pl.ds(L * j, L)] = (
                                rows[e2, pl.ds(L * j, L)] * s)
                    return c2

                lax.fori_loop(0, CHUNK // L, scale, 0)
                pltpu.sync_copy(rows, acc.at[rowv.at[b]], add=True)

                @pl.when(i + 3 < n_chunks)
                def _():
                    start_edges(i + 3, b)
            return carry

        lax.fori_loop(0, n_chunks // NB, outer, 0)
        plsc.subcore_barrier()
        pltpu.sync_copy(acc.at[pl.ds(sid * rpt, rpt)],
                        out_hbm.at[cid, pl.ds(sid * rpt, rpt)])

    return k(x, colp, rowp, cp, disf)


def _linear(p, wt, b2, n, d):
    r = n // 8

    def body(p_ref, w_ref, b_ref, out_ref):
        y = p_ref[0] + p_ref[1]
        out_ref[...] = (
            jnp.dot(y, w_ref[...], preferred_element_type=jnp.float32)
            + b_ref[...])

    return pl.pallas_call(
        body,
        grid=(8,),
        in_specs=[
            pl.BlockSpec((NC, r, d), lambda i: (0, i, 0)),
            pl.BlockSpec((d, d), lambda i: (0, 0)),
            pl.BlockSpec((1, d), lambda i: (0, 0)),
        ],
        out_specs=pl.BlockSpec((r, d), lambda i: (i, 0)),
        out_shape=jax.ShapeDtypeStruct((n, d), jnp.float32),
    )(p, wt, b2)


def kernel(x, edge_index, C, W, b):
    n, d = x.shape
    e = C.shape[0]
    row = edge_index[0]
    col = edge_index[1]

    block = NW * CHUNK
    n_chunks = -(-e // block)
    n_chunks = -(-n_chunks // NB) * NB
    pad = n_chunks * block - e
    if pad:
        zi = jnp.zeros((pad,), jnp.int32)
        row = jnp.concatenate([row, zi])
        col = jnp.concatenate([col, zi])
        cp = jnp.concatenate([C, jnp.zeros((pad,), jnp.float32)])
    else:
        cp = C
    col3 = col.reshape(NW, n_chunks, CHUNK)
    row3 = row.reshape(NW, n_chunks, CHUNK)
    c3 = cp.reshape(NW, n_chunks, CHUNK)

    drows = -(-n // 128)
    drows = -(-drows // NS) * NS
    np_pad = drows * 128
    degp = _deg_partials(col3, c3, np_pad)
    dis = _dis_from_deg(degp.reshape(NC, drows, 128))
    disf = dis.reshape(np_pad)
    np_acc = -(-n // 128) * 128
    p = _prop_partials(x, col, row, cp, disf, n_chunks, np_acc, d)
    out = _linear(p, W.T, b.reshape(1, d), np_acc, d)
    return out[:n]


# speedup vs baseline: 1.4424x; 1.2908x over previous
import functools

import jax
import jax.numpy as jnp
from jax import lax
from jax.experimental import pallas as pl
from jax.experimental.pallas import tpu as pltpu
from jax.experimental.pallas import tpu_sc as plsc

NC = 2
NS = 16
NW = NC * NS
L = 16
CHUNK = 96
NB = 3


def _sc_mesh():
    return plsc.VectorSubcoreMesh(core_axis_name="c", subcore_axis_name="s")


def _deg_partials(col3, c3, ndp):
    n_chunks = col3.shape[1]
    zpt = ndp // NS

    @functools.partial(
        pl.kernel,
        out_type=jax.ShapeDtypeStruct((NC, 1, ndp), jnp.float32),
        mesh=_sc_mesh(),
        scratch_types=[
            pltpu.VMEM((n_chunks, CHUNK), jnp.int32),
            pltpu.VMEM((n_chunks, CHUNK), jnp.float32),
            pltpu.VMEM((zpt,), jnp.float32),
            pltpu.VMEM_SHARED((ndp,), jnp.float32),
        ],
    )
    def k(col_hbm, c_hbm, out_hbm, col2d, c2d, zbuf, deg_acc):
        cid = lax.axis_index("c")
        sid = lax.axis_index("s")
        wid = cid * NS + sid
        zero16 = jnp.zeros((L,), jnp.float32)

        def zz(i, carry):
            zbuf[pl.ds(i * L, L)] = zero16
            return carry

        lax.fori_loop(0, zpt // L, zz, 0)
        pltpu.sync_copy(zbuf, deg_acc.at[pl.ds(sid * zpt, zpt)])
        pltpu.sync_copy(col_hbm.at[wid], col2d)
        pltpu.sync_copy(c_hbm.at[wid], c2d)
        plsc.subcore_barrier()

        def body(i, carry):
            pltpu.sync_copy(c2d.at[i], deg_acc.at[col2d.at[i]], add=True)
            return carry

        lax.fori_loop(0, n_chunks, body, 0)
        plsc.subcore_barrier()
        pltpu.sync_copy(deg_acc.at[pl.ds(sid * zpt, zpt)],
                        out_hbm.at[cid, 0, pl.ds(sid * zpt, zpt)])

    return k(col3, c3)


def _dis_from_deg(degp):
    def body(deg_ref, out_ref):
        d = jnp.sum(deg_ref[...], axis=0)
        out_ref[...] = jnp.where(
            d > 0, lax.rsqrt(jnp.maximum(d, 1e-30)), 0.0)

    return pl.pallas_call(
        body,
        out_shape=jax.ShapeDtypeStruct(degp.shape[1:], jnp.float32),
    )(degp)


def _prop_partials(x, colp, rowp, cp, disf, n_chunks, n, d):
    rpt = n // NS
    zrows = CHUNK
    ndp = disf.shape[0]

    @functools.partial(
        pl.kernel,
        out_type=jax.ShapeDtypeStruct((NC, n, d), jnp.float32),
        mesh=_sc_mesh(),
        scratch_types=(
            [pltpu.VMEM((NB, CHUNK), jnp.int32),
             pltpu.VMEM((NB, CHUNK), jnp.int32),
             pltpu.VMEM((NB, CHUNK), jnp.float32),
             pltpu.VMEM((NB, CHUNK), jnp.float32),
             pltpu.VMEM((NB, CHUNK), jnp.float32)]
            + [pltpu.VMEM((CHUNK, 128), jnp.float32) for _ in range(NB)]
            + [pltpu.VMEM_SHARED((n, d), jnp.float32)]
            + [pltpu.SemaphoreType.DMA for _ in range(2 * NB)]
        ),
    )
    def k(x_hbm, col_hbm, row_hbm, c_hbm, dis_hbm, out_hbm, *refs):
        refs = list(refs)
        colv, rowv, cvb, dcv, drv = refs[:5]
        rbufs = refs[5:5 + NB]
        acc = refs[5 + NB]
        esems = refs[6 + NB:6 + 2 * NB]
        sems = refs[6 + 2 * NB:6 + 3 * NB]
        cid = lax.axis_index("c")
        sid = lax.axis_index("s")
        wid = cid * NS + sid
        zero16 = jnp.zeros((L,), jnp.float32)
        ept = n_chunks * CHUNK

        def zr(i, carry):
            for j in range(d // L):
                rbufs[0][i, pl.ds(L * j, L)] = zero16
            return carry

        lax.fori_loop(0, zrows, zr, 0)

        def zacc(i, carry):
            pltpu.sync_copy(rbufs[0],
                            acc.at[pl.ds(sid * rpt + i * zrows, zrows)])
            return carry

        lax.fori_loop(0, rpt // zrows, zacc, 0)
        rem = rpt % zrows
        if rem:
            pltpu.sync_copy(
                rbufs[0].at[pl.ds(0, rem)],
                acc.at[pl.ds(sid * rpt + (rpt // zrows) * zrows, rem)])
        plsc.subcore_barrier()

        def start_edges(i, buf):
            base = wid * ept + i * CHUNK
            pltpu.async_copy(col_hbm.at[pl.ds(base, CHUNK)],
                             colv.at[buf], esems[buf])
            pltpu.async_copy(row_hbm.at[pl.ds(base, CHUNK)],
                             rowv.at[buf], esems[buf])
            pltpu.async_copy(c_hbm.at[pl.ds(base, CHUNK)],
                             cvb.at[buf], esems[buf])

        def wait_edges(i, buf):
            base = wid * ept + i * CHUNK
            pltpu.make_async_copy(col_hbm.at[pl.ds(base, CHUNK)],
                                  colv.at[buf], esems[buf]).wait()
            pltpu.make_async_copy(row_hbm.at[pl.ds(base, CHUNK)],
                                  rowv.at[buf], esems[buf]).wait()
            pltpu.make_async_copy(c_hbm.at[pl.ds(base, CHUNK)],
                                  cvb.at[buf], esems[buf]).wait()

        def start_gathers(buf):
            pltpu.async_copy(x_hbm.at[colv.at[buf]], rbufs[buf], sems[buf])
            pltpu.async_copy(dis_hbm.at[colv.at[buf]], dcv.at[buf], sems[buf])
            pltpu.async_copy(dis_hbm.at[rowv.at[buf]], drv.at[buf], sems[buf])

        def wait_gathers(buf):
            pltpu.make_async_copy(x_hbm.at[colv.at[buf]],
                                  rbufs[buf], sems[buf]).wait()
            pltpu.make_async_copy(dis_hbm.at[colv.at[buf]],
                                  dcv.at[buf], sems[buf]).wait()
            pltpu.make_async_copy(dis_hbm.at[rowv.at[buf]],
                                  drv.at[buf], sems[buf]).wait()

        start_edges(0, 0)
        wait_edges(0, 0)
        start_gathers(0)
        start_edges(1, 1)
        wait_edges(1, 1)
        start_gathers(1)
        start_edges(2, 2)

        def outer(io, carry):
            for b in range(NB):
                i = io * NB + b
                b2 = (b + 2) % NB

                @pl.when(i + 2 < n_chunks)
                def _():
                    wait_edges(i + 2, b2)
                    start_gathers(b2)

                wait_gathers(b)
                rows = rbufs[b]

                def scale(g, c2):
                    svec = (cvb[b, pl.ds(g * L, L)]
                            * dcv[b, pl.ds(g * L, L)]
                            * drv[b, pl.ds(g * L, L)])
                    for kq in range(L):
                        s = svec[kq]
                        e2 = g * L + kq
                        for j in range(d // L):
                            rows[e2, pl.ds(L * j, L)] = (
                                rows[e2, pl.ds(L * j, L)] * s)
                    return c2

                lax.fori_loop(0, CHUNK // L, scale, 0)
                pltpu.sync_copy(rows, acc.at[rowv.at[b]], add=True)

                @pl.when(i + 3 < n_chunks)
                def _():
                    start_edges(i + 3, b)
            return carry

        lax.fori_loop(0, n_chunks // NB, outer, 0)
        plsc.subcore_barrier()
        pltpu.sync_copy(acc.at[pl.ds(sid * rpt, rpt)],
                        out_hbm.at[cid, pl.ds(sid * rpt, rpt)])

    return k(x, colp, rowp, cp, disf)


def _linear(p, wt, b2, n, d):
    r = n // 8

    def body(p_ref, w_ref, b_ref, out_ref):
        y = p_ref[0] + p_ref[1]
        out_ref[...] = (
            jnp.dot(y, w_ref[...], preferred_element_type=jnp.float32)
            + b_ref[...])

    return pl.pallas_call(
        body,
        grid=(8,),
        in_specs=[
            pl.BlockSpec((NC, r, d), lambda i: (0, i, 0)),
            pl.BlockSpec((d, d), lambda i: (0, 0)),
            pl.BlockSpec((1, d), lambda i: (0, 0)),
        ],
        out_specs=pl.BlockSpec((r, d), lambda i: (i, 0)),
        out_shape=jax.ShapeDtypeStruct((n, d), jnp.float32),
    )(p, wt, b2)


def kernel(x, edge_index, C, W, b):
    n, d = x.shape
    e = C.shape[0]
    row = edge_index[0]
    col = edge_index[1]

    block = NW * CHUNK
    n_chunks = -(-e // block)
    n_chunks = -(-n_chunks // NB) * NB
    pad = n_chunks * block - e
    if pad:
        zi = jnp.zeros((pad,), jnp.int32)
        row = jnp.concatenate([row, zi])
        col = jnp.concatenate([col, zi])
        cp = jnp.concatenate([C, jnp.zeros((pad,), jnp.float32)])
    else:
        cp = C
    col3 = col.reshape(NW, n_chunks, CHUNK)
    row3 = row.reshape(NW, n_chunks, CHUNK)
    c3 = cp.reshape(NW, n_chunks, CHUNK)

    drows = -(-n // 128)
    drows = -(-drows // NS) * NS
    np_pad = drows * 128
    degp = _deg_partials(col3, c3, np_pad)
    dis = _dis_from_deg(degp.reshape(NC, drows, 128))
    disf = dis.reshape(np_pad)
    np_acc = -(-n // 128) * 128
    p = _prop_partials(x, col, row, cp, disf, n_chunks, np_acc, d)
    out = _linear(p, W.T, b.reshape(1, d), np_acc, d)
    return out[:n]


# speedup vs baseline: 1.4615x; 1.0133x over previous
import functools

import jax
import jax.numpy as jnp
from jax import lax
from jax.experimental import pallas as pl
from jax.experimental.pallas import tpu as pltpu
from jax.experimental.pallas import tpu_sc as plsc

NC = 2
NS = 16
NW = NC * NS
L = 16
CHUNK = 112
NB = 3


def _sc_mesh():
    return plsc.VectorSubcoreMesh(core_axis_name="c", subcore_axis_name="s")


def _deg_partials(col3, c3, ndp):
    n_chunks = col3.shape[1]
    zpt = ndp // NS

    @functools.partial(
        pl.kernel,
        out_type=jax.ShapeDtypeStruct((NC, 1, ndp), jnp.float32),
        mesh=_sc_mesh(),
        scratch_types=[
            pltpu.VMEM((n_chunks, CHUNK), jnp.int32),
            pltpu.VMEM((n_chunks, CHUNK), jnp.float32),
            pltpu.VMEM((zpt,), jnp.float32),
            pltpu.VMEM_SHARED((ndp,), jnp.float32),
        ],
    )
    def k(col_hbm, c_hbm, out_hbm, col2d, c2d, zbuf, deg_acc):
        cid = lax.axis_index("c")
        sid = lax.axis_index("s")
        wid = cid * NS + sid
        zero16 = jnp.zeros((L,), jnp.float32)

        def zz(i, carry):
            zbuf[pl.ds(i * L, L)] = zero16
            return carry

        lax.fori_loop(0, zpt // L, zz, 0)
        pltpu.sync_copy(zbuf, deg_acc.at[pl.ds(sid * zpt, zpt)])
        pltpu.sync_copy(col_hbm.at[wid], col2d)
        pltpu.sync_copy(c_hbm.at[wid], c2d)
        plsc.subcore_barrier()

        def body(i, carry):
            pltpu.sync_copy(c2d.at[i], deg_acc.at[col2d.at[i]], add=True)
            return carry

        lax.fori_loop(0, n_chunks, body, 0)
        plsc.subcore_barrier()
        pltpu.sync_copy(deg_acc.at[pl.ds(sid * zpt, zpt)],
                        out_hbm.at[cid, 0, pl.ds(sid * zpt, zpt)])

    return k(col3, c3)


def _dis_from_deg(degp):
    def body(deg_ref, out_ref):
        d = jnp.sum(deg_ref[...], axis=0)
        out_ref[...] = jnp.where(
            d > 0, lax.rsqrt(jnp.maximum(d, 1e-30)), 0.0)

    return pl.pallas_call(
        body,
        out_shape=jax.ShapeDtypeStruct(degp.shape[1:], jnp.float32),
    )(degp)


def _prop_partials(x, colp, rowp, cp, disf, n_chunks, n, d):
    rpt = n // NS
    zrows = CHUNK
    ndp = disf.shape[0]

    @functools.partial(
        pl.kernel,
        out_type=jax.ShapeDtypeStruct((NC, n, d), jnp.float32),
        mesh=_sc_mesh(),
        scratch_types=(
            [pltpu.VMEM((NB, CHUNK), jnp.int32),
             pltpu.VMEM((NB, CHUNK), jnp.int32),
             pltpu.VMEM((NB, CHUNK), jnp.float32),
             pltpu.VMEM((NB, CHUNK), jnp.float32),
             pltpu.VMEM((NB, CHUNK), jnp.float32)]
            + [pltpu.VMEM((CHUNK, 128), jnp.float32) for _ in range(NB)]
            + [pltpu.VMEM_SHARED((n, d), jnp.float32)]
            + [pltpu.SemaphoreType.DMA for _ in range(2 * NB)]
        ),
    )
    def k(x_hbm, col_hbm, row_hbm, c_hbm, dis_hbm, out_hbm, *refs):
        refs = list(refs)
        colv, rowv, cvb, dcv, drv = refs[:5]
        rbufs = refs[5:5 + NB]
        acc = refs[5 + NB]
        esems = refs[6 + NB:6 + 2 * NB]
        sems = refs[6 + 2 * NB:6 + 3 * NB]
        cid = lax.axis_index("c")
        sid = lax.axis_index("s")
        wid = cid * NS + sid
        zero16 = jnp.zeros((L,), jnp.float32)
        ept = n_chunks * CHUNK

        def zr(i, carry):
            for j in range(d // L):
                rbufs[0][i, pl.ds(L * j, L)] = zero16
            return carry

        lax.fori_loop(0, zrows, zr, 0)

        def zacc(i, carry):
            pltpu.sync_copy(rbufs[0],
                            acc.at[pl.ds(sid * rpt + i * zrows, zrows)])
            return carry

        lax.fori_loop(0, rpt // zrows, zacc, 0)
        rem = rpt % zrows
        if rem:
            pltpu.sync_copy(
                rbufs[0].at[pl.ds(0, rem)],
                acc.at[pl.ds(sid * rpt + (rpt // zrows) * zrows, rem)])
        plsc.subcore_barrier()

        def start_edges(i, buf):
            base = wid * ept + i * CHUNK
            pltpu.async_copy(col_hbm.at[pl.ds(base, CHUNK)],
                             colv.at[buf], esems[buf])
            pltpu.async_copy(row_hbm.at[pl.ds(base, CHUNK)],
                             rowv.at[buf], esems[buf])
            pltpu.async_copy(c_hbm.at[pl.ds(base, CHUNK)],
                             cvb.at[buf], esems[buf])

        def wait_edges(i, buf):
            base = wid * ept + i * CHUNK
            pltpu.make_async_copy(col_hbm.at[pl.ds(base, CHUNK)],
                                  colv.at[buf], esems[buf]).wait()
            pltpu.make_async_copy(row_hbm.at[pl.ds(base, CHUNK)],
                                  rowv.at[buf], esems[buf]).wait()
            pltpu.make_async_copy(c_hbm.at[pl.ds(base, CHUNK)],
                                  cvb.at[buf], esems[buf]).wait()

        def start_gathers(buf):
            pltpu.async_copy(x_hbm.at[colv.at[buf]], rbufs[buf], sems[buf])
            pltpu.async_copy(dis_hbm.at[colv.at[buf]], dcv.at[buf], sems[buf])
            pltpu.async_copy(dis_hbm.at[rowv.at[buf]], drv.at[buf], sems[buf])

        def wait_gathers(buf):
            pltpu.make_async_copy(x_hbm.at[colv.at[buf]],
                                  rbufs[buf], sems[buf]).wait()
            pltpu.make_async_copy(dis_hbm.at[colv.at[buf]],
                                  dcv.at[buf], sems[buf]).wait()
            pltpu.make_async_copy(dis_hbm.at[rowv.at[buf]],
                                  drv.at[buf], sems[buf]).wait()

        start_edges(0, 0)
        wait_edges(0, 0)
        start_gathers(0)
        start_edges(1, 1)
        wait_edges(1, 1)
        start_gathers(1)
        start_edges(2, 2)

        def outer(io, carry):
            for b in range(NB):
                i = io * NB + b
                b2 = (b + 2) % NB

                @pl.when(i + 2 < n_chunks)
                def _():
                    wait_edges(i + 2, b2)
                    start_gathers(b2)

                wait_gathers(b)
                rows = rbufs[b]

                def scale(g, c2):
                    svec = (cvb[b, pl.ds(g * L, L)]
                            * dcv[b, pl.ds(g * L, L)]
                            * drv[b, pl.ds(g * L, L)])
                    for kq in range(L):
                        s = svec[kq]
                        e2 = g * L + kq
                        for j in range(d // L):
                            rows[e2, pl.ds(L * j, L)] = (
                                rows[e2, pl.ds(L * j, L)] * s)
                    return c2

                lax.fori_loop(0, CHUNK // L, scale, 0)
                pltpu.sync_copy(rows, acc.at[rowv.at[b]], add=True)

                @pl.when(i + 3 < n_chunks)
                def _():
                    start_edges(i + 3, b)
            return carry

        lax.fori_loop(0, n_chunks // NB, outer, 0)
        plsc.subcore_barrier()
        pltpu.sync_copy(acc.at[pl.ds(sid * rpt, rpt)],
                        out_hbm.at[cid, pl.ds(sid * rpt, rpt)])

    return k(x, colp, rowp, cp, disf)


def _linear(p, wt, b2, n, d):
    r = n // 8

    def body(p_ref, w_ref, b_ref, out_ref):
        y = p_ref[0] + p_ref[1]
        out_ref[...] = (
            jnp.dot(y, w_ref[...], preferred_element_type=jnp.float32)
            + b_ref[...])

    return pl.pallas_call(
        body,
        grid=(8,),
        in_specs=[
            pl.BlockSpec((NC, r, d), lambda i: (0, i, 0)),
            pl.BlockSpec((d, d), lambda i: (0, 0)),
            pl.BlockSpec((1, d), lambda i: (0, 0)),
        ],
        out_specs=pl.BlockSpec((r, d), lambda i: (i, 0)),
        out_shape=jax.ShapeDtypeStruct((n, d), jnp.float32),
    )(p, wt, b2)


def kernel(x, edge_index, C, W, b):
    n, d = x.shape
    e = C.shape[0]
    row = edge_index[0]
    col = edge_index[1]

    block = NW * CHUNK
    n_chunks = -(-e // block)
    n_chunks = -(-n_chunks // NB) * NB
    pad = n_chunks * block - e
    if pad:
        zi = jnp.zeros((pad,), jnp.int32)
        row = jnp.concatenate([row, zi])
        col = jnp.concatenate([col, zi])
        cp = jnp.concatenate([C, jnp.zeros((pad,), jnp.float32)])
    else:
        cp = C
    col3 = col.reshape(NW, n_chunks, CHUNK)
    row3 = row.reshape(NW, n_chunks, CHUNK)
    c3 = cp.reshape(NW, n_chunks, CHUNK)

    drows = -(-n // 128)
    drows = -(-drows // NS) * NS
    np_pad = drows * 128
    degp = _deg_partials(col3, c3, np_pad)
    dis = _dis_from_deg(degp.reshape(NC, drows, 128))
    disf = dis.reshape(np_pad)
    np_acc = -(-n // 128) * 128
    p = _prop_partials(x, col, row, cp, disf, n_chunks, np_acc, d)
    out = _linear(p, W.T, b.reshape(1, d), np_acc, d)
    return out[:n]
